# baseline (speedup 1.0000x reference)
import functools

import jax
import jax.numpy as jnp
from jax import lax
from jax.experimental import pallas as pl
from jax.experimental.pallas import tpu as pltpu
from jax.experimental.pallas import tpu_sc as plsc

_BATCH = 16384
_DIM = 64
_LANES = 16
_NCORES = 2
_NSUB = 16
_NWORKERS = _NCORES * _NSUB
_BPW = _BATCH // _NWORKERS
_CHUNK = 128
_NCHUNK = _BPW // _CHUNK
_NGROUP = _CHUNK // _LANES
_NCHB = _DIM // _LANES


def _build():
    mesh = plsc.VectorSubcoreMesh(core_axis_name="c", subcore_axis_name="s")

    @functools.partial(
        pl.kernel,
        out_type=jax.ShapeDtypeStruct((_BATCH,), jnp.float32),
        mesh=mesh,
        scratch_types=[
            pltpu.VMEM((_NCHUNK, _CHUNK), jnp.int32),
            pltpu.VMEM((_BPW,), jnp.int32),
            pltpu.VMEM((_BPW, _DIM), jnp.float32),
            pltpu.VMEM((2 * _DIM,), jnp.float32),
            pltpu.VMEM((_LANES * _LANES,), jnp.float32),
            pltpu.VMEM((_BPW,), jnp.float32),
        ] + [pltpu.SemaphoreType.DMA] * _NCHUNK,
        compiler_params=pltpu.CompilerParams(
            needs_layout_passes=False, use_tc_tiling_on_sc=False),
    )
    def scores_kernel(idx_hbm, prf_hbm, table_hbm, ptab_hbm, out_hbm,
                      idx_v, prf_v, rows_v, ptab_v, tmp_v, sc_v, *sems):
        wid = lax.axis_index("s") * _NCORES + lax.axis_index("c")
        base = wid * _BPW

        for c in range(_NCHUNK):
            pltpu.sync_copy(idx_hbm.at[pl.ds(base + c * _CHUNK, _CHUNK)],
                            idx_v.at[c])
        gathers = []
        for c in range(_NCHUNK):
            gathers.append(
                pltpu.async_copy(table_hbm.at[idx_v.at[c]],
                                 rows_v.at[pl.ds(c * _CHUNK, _CHUNK)],
                                 sems[c]))
        pltpu.sync_copy(prf_hbm.at[pl.ds(base, _BPW)], prf_v)
        pltpu.sync_copy(ptab_hbm, ptab_v)

        p0 = [ptab_v[pl.ds(j * _LANES, _LANES)] for j in range(_NCHB)]
        p1 = [ptab_v[pl.ds(_DIM + j * _LANES, _LANES)] for j in range(_NCHB)]
        col = lax.broadcasted_iota(jnp.int32, (_LANES,), 0) * _LANES

        for c in range(_NCHUNK):
            gathers[c].wait()

            def group_body(g, _, c=c):
                rbase = c * _CHUNK + g * _LANES
                pvec = prf_v[pl.ds(rbase, _LANES)]
                for r in range(_LANES):
                    row = rbase + r
                    p = pvec[r]
                    prod = None
                    for j in range(_NCHB):
                        rc = rows_v[row, pl.ds(j * _LANES, _LANES)]
                        w = jnp.where(p > 0, p1[j], p0[j])
                        prod = rc * w if prod is None else prod + rc * w
                    plsc.store_scatter(tmp_v, [col + r], prod)
                acc = tmp_v[pl.ds(0, _LANES)]
                for d in range(1, _LANES):
                    acc = acc + tmp_v[pl.ds(d * _LANES, _LANES)]
                sc_v[pl.ds(rbase, _LANES)] = acc
                return 0

            lax.fori_loop(0, _NGROUP, group_body, 0)

        pltpu.sync_copy(sc_v, out_hbm.at[pl.ds(base, _BPW)])

    return scores_kernel


_scores_kernel = _build()


def kernel(article_indices, preferences, article_table, preference_table):
    idx = article_indices.astype(jnp.int32)
    prf = preferences.astype(jnp.int32)
    ptab = preference_table.reshape(-1).astype(jnp.float32)
    return _scores_kernel(idx, prf, article_table, ptab)
